# baseline (speedup 1.0000x reference)
import functools

import jax
import jax.numpy as jnp
from jax import lax
from jax.experimental import pallas as pl
from jax.experimental.pallas import tpu as pltpu
from jax.experimental.pallas import tpu_sc as plsc

B = 32
S = 2048
H = 64
L = 16

_info = plsc.get_sparse_core_info()
NC, NS = _info.num_cores, _info.num_subcores
NW = NC * NS
S_PER_W = S // NW


def _body(ids_hbm, tok_hbm, pos_hbm, out_hbm, idx_v, pos_v, rows_v, sem):
    wid = lax.axis_index("s") * NC + lax.axis_index("c")
    s0 = wid * S_PER_W

    pltpu.sync_copy(pos_hbm.at[pl.ds(s0, S_PER_W)], pos_v)

    def per_batch(b, carry):
        pltpu.sync_copy(ids_hbm.at[pl.ds(b * S + s0, S_PER_W)], idx_v)
        pltpu.async_copy(tok_hbm.at[idx_v], rows_v, sem).wait()

        def add_row(k, c):
            i = k // (H // L)
            j = (k % (H // L)) * L
            plsc.addupdate(rows_v.at[i, pl.ds(j, L)], pos_v[i, pl.ds(j, L)])
            return c

        lax.fori_loop(0, S_PER_W * (H // L), add_row, 0)

        pltpu.sync_copy(rows_v, out_hbm.at[b, pl.ds(s0, S_PER_W)])
        return carry

    lax.fori_loop(0, B, per_batch, 0)


def kernel(input_ids, token_table, pos_table):
    mesh = plsc.VectorSubcoreMesh(core_axis_name="c", subcore_axis_name="s")
    k = functools.partial(
        pl.kernel,
        mesh=mesh,
        out_type=jax.ShapeDtypeStruct((B, S, H), jnp.float32),
        scratch_types=[
            pltpu.VMEM((S_PER_W,), jnp.int32),
            pltpu.VMEM((S_PER_W, H), jnp.float32),
            pltpu.VMEM((S_PER_W, H), jnp.float32),
            pltpu.SemaphoreType.DMA,
        ],
        compiler_params=pltpu.CompilerParams(use_tc_tiling_on_sc=False),
    )(_body)
    return k(input_ids.reshape(B * S), token_table, pos_table)


# speedup vs baseline: 1.0887x; 1.0887x over previous
import functools

import jax
import jax.numpy as jnp
from jax import lax
from jax.experimental import pallas as pl
from jax.experimental.pallas import tpu as pltpu
from jax.experimental.pallas import tpu_sc as plsc

B = 32
S = 2048
H = 64
L = 16

_info = plsc.get_sparse_core_info()
NC, NS = _info.num_cores, _info.num_subcores
NW = NC * NS
S_PER_W = S // NW
K = 4
NSUP = B // K
NBUF = 4


def _body(ids_hbm, tok_hbm, pos_hbm, out_hbm, idx_v, pos_v, rows, sems_g,
          sems_w, sem_p):
    wid = lax.axis_index("s") * NC + lax.axis_index("c")
    s0 = wid * S_PER_W

    cp = pltpu.async_copy(pos_hbm.at[pl.ds(s0, S_PER_W)], pos_v, sem_p)
    ci = pltpu.async_copy(ids_hbm.at[:, pl.ds(s0, S_PER_W)], idx_v, sem_p)
    cp.wait()
    ci.wait()

    def fire_gathers(step):
        p = step % NBUF
        return [
            pltpu.async_copy(
                tok_hbm.at[idx_v.at[step * K + j]], rows[p].at[j], sems_g[p])
            for j in range(K)
        ]

    def add_pos(p):
        def body(i, c):
            for j in range(K):
                for k in range(H // L):
                    plsc.addupdate(rows[p].at[j, i, pl.ds(k * L, L)],
                                   pos_v[i, pl.ds(k * L, L)])
            return c

        lax.fori_loop(0, S_PER_W, body, 0)

    gathers = {0: fire_gathers(0), 1: fire_gathers(1)}
    writes = {}
    for s in range(NSUP):
        p = s % NBUF
        for c in gathers.pop(s):
            c.wait()
        add_pos(p)
        writes[s] = pltpu.async_copy(
            rows[p], out_hbm.at[pl.ds(s * K, K), pl.ds(s0, S_PER_W)],
            sems_w[p])
        nxt = s + 2
        if nxt < NSUP:
            if nxt >= NBUF:
                writes.pop(nxt - NBUF).wait()
            gathers[nxt] = fire_gathers(nxt)
    for s in sorted(writes):
        writes[s].wait()


def kernel(input_ids, token_table, pos_table):
    mesh = plsc.VectorSubcoreMesh(core_axis_name="c", subcore_axis_name="s")

    def body(ids_hbm, tok_hbm, pos_hbm, out_hbm, idx_v, pos_v, r0, r1, r2, r3,
             g0, g1, g2, g3, w0, w1, w2, w3, sem_p):
        _body(ids_hbm, tok_hbm, pos_hbm, out_hbm, idx_v, pos_v,
              [r0, r1, r2, r3], [g0, g1, g2, g3], [w0, w1, w2, w3], sem_p)

    k = functools.partial(
        pl.kernel,
        mesh=mesh,
        out_type=jax.ShapeDtypeStruct((B, S, H), jnp.float32),
        scratch_types=(
            [pltpu.VMEM((B, S_PER_W), jnp.int32),
             pltpu.VMEM((S_PER_W, H), jnp.float32)]
            + [pltpu.VMEM((K, S_PER_W, H), jnp.float32) for _ in range(NBUF)]
            + [pltpu.SemaphoreType.DMA for _ in range(2 * NBUF + 1)]
        ),
        compiler_params=pltpu.CompilerParams(use_tc_tiling_on_sc=False),
    )(body)
    return k(input_ids, token_table, pos_table)
